# baseline (speedup 1.0000x reference)
import jax
import jax.numpy as jnp
from jax.experimental import pallas as pl


def _add_kernel(x_ref, t_ref, o_ref):
    o_ref[...] = x_ref[...] + t_ref[...][None, :, :]


def kernel(x, table):
    batch, num_patches, proj_dim = x.shape
    block_b = 1
    grid = (batch // block_b,)
    return pl.pallas_call(
        _add_kernel,
        grid=grid,
        in_specs=[
            pl.BlockSpec((block_b, num_patches, proj_dim), lambda b: (b, 0, 0)),
            pl.BlockSpec((num_patches, proj_dim), lambda b: (0, 0)),
        ],
        out_specs=pl.BlockSpec((block_b, num_patches, proj_dim), lambda b: (b, 0, 0)),
        out_shape=jax.ShapeDtypeStruct(x.shape, x.dtype),
    )(x, table)


# speedup vs baseline: 1.0312x; 1.0312x over previous
import jax
import jax.numpy as jnp
from jax.experimental import pallas as pl


def _add_kernel(x_ref, t_ref, o_ref):
    o_ref[...] = x_ref[...] + t_ref[...][None, :, :]


def kernel(x, table):
    batch, num_patches, proj_dim = x.shape
    block_b = 2
    grid = (batch // block_b,)
    return pl.pallas_call(
        _add_kernel,
        grid=grid,
        in_specs=[
            pl.BlockSpec((block_b, num_patches, proj_dim), lambda b: (b, 0, 0)),
            pl.BlockSpec((num_patches, proj_dim), lambda b: (0, 0)),
        ],
        out_specs=pl.BlockSpec((block_b, num_patches, proj_dim), lambda b: (b, 0, 0)),
        out_shape=jax.ShapeDtypeStruct(x.shape, x.dtype),
    )(x, table)


# speedup vs baseline: 1.0440x; 1.0124x over previous
import jax
import jax.numpy as jnp
from jax.experimental import pallas as pl


def _add_kernel(x_ref, t_ref, o_ref):
    o_ref[...] = x_ref[...] + t_ref[...][None, :, :]


def kernel(x, table):
    batch, num_patches, proj_dim = x.shape
    block_b = 4
    grid = (batch // block_b,)
    return pl.pallas_call(
        _add_kernel,
        grid=grid,
        in_specs=[
            pl.BlockSpec((block_b, num_patches, proj_dim), lambda b: (b, 0, 0)),
            pl.BlockSpec((num_patches, proj_dim), lambda b: (0, 0)),
        ],
        out_specs=pl.BlockSpec((block_b, num_patches, proj_dim), lambda b: (b, 0, 0)),
        out_shape=jax.ShapeDtypeStruct(x.shape, x.dtype),
    )(x, table)
